# Initial kernel scaffold; baseline (speedup 1.0000x reference)
import jax
import jax.numpy as jnp
from jax.experimental import pallas as pl


def kernel(x_node, x_edge, edge_index, b_mask, W1e, b1e, W2e, b2e, W1n, b1n, W2n, b2n):
    raise NotImplementedError("write your pallas kernel here")


# speedup vs baseline: 2.4075x; 2.4075x over previous
import functools

import jax
import jax.numpy as jnp
from jax import lax
from jax.experimental import pallas as pl
from jax.experimental.pallas import tpu as pltpu
from jax.experimental.pallas import tpu_sc as plsc

_NC = 2
_NS = 16
_NW = _NC * _NS


def _tc_tables(x_node, w1ea, w1eb, b1e):
    n_nodes, d = x_node.shape
    blk = 5000
    grid = (n_nodes // blk,)

    def body(xn_ref, wa_ref, wb_ref, b_ref, pa_ref, pb_ref):
        xn = xn_ref[...]
        f32 = jnp.float32
        pa_ref[...] = (jnp.dot(xn, wa_ref[...], preferred_element_type=f32)
                       + b_ref[0:1, :])
        pb_ref[...] = jnp.dot(xn, wb_ref[...], preferred_element_type=f32)

    return pl.pallas_call(
        body,
        grid=grid,
        in_specs=[
            pl.BlockSpec((blk, d), lambda i: (i, 0)),
            pl.BlockSpec((d, 32), lambda i: (0, 0)),
            pl.BlockSpec((d, 32), lambda i: (0, 0)),
            pl.BlockSpec((8, 32), lambda i: (0, 0)),
        ],
        out_specs=[
            pl.BlockSpec((blk, 32), lambda i: (i, 0)),
            pl.BlockSpec((blk, 32), lambda i: (i, 0)),
        ],
        out_shape=[
            jax.ShapeDtypeStruct((n_nodes, 32), jnp.float32),
            jax.ShapeDtypeStruct((n_nodes, 32), jnp.float32),
        ],
        compiler_params=pltpu.CompilerParams(
            dimension_semantics=("arbitrary",),
        ),
    )(x_node, w1ea, w1eb, b1e)


def _sc_gather(x_node, pa, pb, src, dst, b_mask):
    n_nodes, d = x_node.shape
    n_edges = src.shape[0]
    per_w = n_edges // _NW
    chunk = 80
    n_chunks = per_w // chunk
    ngrp = chunk // 16
    mesh = plsc.VectorSubcoreMesh(core_axis_name="c", subcore_axis_name="s")

    @functools.partial(
        pl.kernel,
        out_type=(
            jax.ShapeDtypeStruct((n_edges, d), jnp.float32),
            jax.ShapeDtypeStruct((n_edges, d), jnp.float32),
        ),
        mesh=mesh,
        scratch_types=[
            pltpu.VMEM((2, chunk), jnp.int32),
            pltpu.VMEM((2, chunk), jnp.int32),
            pltpu.VMEM((2, chunk), jnp.int32),
            pltpu.VMEM((2, chunk, d), jnp.float32),
            pltpu.VMEM((2, chunk, d), jnp.float32),
            pltpu.VMEM((2, chunk, d), jnp.float32),
            pltpu.SemaphoreType.DMA,
            pltpu.SemaphoreType.DMA,
            pltpu.SemaphoreType.DMA,
        ],
        compiler_params=pltpu.CompilerParams(use_tc_tiling_on_sc=False),
    )
    def gather_kernel(xn, pa_h, pb_h, src_h, dst_h, bm_h, u1_out, xd_out,
                      i1b, i2b, mb, ga, gb, gd, gsem, gsem2, osem):
        wid = lax.axis_index("s") * _NC + lax.axis_index("c")
        base0 = wid * per_w

        def fire(j, slot):
            base = base0 + j * chunk
            pltpu.sync_copy(src_h.at[pl.ds(base, chunk)], i1b.at[slot])
            pltpu.sync_copy(dst_h.at[pl.ds(base, chunk)], i2b.at[slot])
            pltpu.sync_copy(bm_h.at[pl.ds(base, chunk)], mb.at[slot])
            for k in range(ngrp):
                sl = pl.ds(k * 16, 16)
                s_v = i1b[slot, sl]
                d_v = i2b[slot, sl]
                fwd = mb[slot, sl] == 0
                i1b[slot, sl] = jnp.where(fwd, s_v, d_v)
                mb[slot, sl] = jnp.where(fwd, d_v, s_v)
            pltpu.async_copy(pa_h.at[i1b.at[slot]], ga.at[slot], gsem)
            pltpu.async_copy(pb_h.at[mb.at[slot]], gb.at[slot], gsem)
            pltpu.async_copy(xn.at[i2b.at[slot]], gd.at[slot], gsem2)

        def drain_gather(slot):
            pltpu.make_async_copy(pa_h.at[i1b.at[slot]], ga.at[slot], gsem).wait()
            pltpu.make_async_copy(pb_h.at[mb.at[slot]], gb.at[slot], gsem).wait()
            pltpu.make_async_copy(xn.at[i2b.at[slot]], gd.at[slot], gsem2).wait()

        fire(0, 0)

        def body(j, carry):
            slot = lax.rem(j, 2)
            nslot = lax.rem(j + 1, 2)

            @pl.when(j >= 1)
            def _():
                base_p = base0 + (j - 1) * chunk
                pltpu.make_async_copy(
                    ga.at[nslot], u1_out.at[pl.ds(base_p, chunk)], osem
                ).wait()
                pltpu.make_async_copy(
                    gd.at[nslot], xd_out.at[pl.ds(base_p, chunk)], osem
                ).wait()

            @pl.when(j + 1 < n_chunks)
            def _():
                fire(j + 1, nslot)

            drain_gather(slot)
            def addrow(r, carry2):
                ga[slot, r, pl.ds(0, 16)] = (
                    ga[slot, r, pl.ds(0, 16)] + gb[slot, r, pl.ds(0, 16)])
                ga[slot, r, pl.ds(16, 16)] = (
                    ga[slot, r, pl.ds(16, 16)] + gb[slot, r, pl.ds(16, 16)])
                return carry2

            lax.fori_loop(0, chunk, addrow, 0)
            base = base0 + j * chunk
            pltpu.async_copy(ga.at[slot], u1_out.at[pl.ds(base, chunk)], osem)
            pltpu.async_copy(gd.at[slot], xd_out.at[pl.ds(base, chunk)], osem)
            return carry

        lax.fori_loop(0, n_chunks, body, 0)
        base_l = base0 + (n_chunks - 1) * chunk
        slot_l = (n_chunks - 1) % 2
        pltpu.make_async_copy(
            ga.at[slot_l], u1_out.at[pl.ds(base_l, chunk)], osem
        ).wait()
        pltpu.make_async_copy(
            gd.at[slot_l], xd_out.at[pl.ds(base_l, chunk)], osem
        ).wait()

    return gather_kernel(x_node, pa, pb, src, dst, b_mask)


def _tc_mlp(u1, xd, x_edge,
            w1ec, w2e, b2e, w1na, w1nb, b1n, w2n, b2n):
    n_edges, d = u1.shape
    de = x_edge.shape[1]
    blk = 3200
    grid = (n_edges // blk,)

    def body(u1_ref, xd_ref, xe_ref,
             w1ec_r, w2e_r, b2e_r, w1na_r, w1nb_r, b1n_r, w2n_r, b2n_r,
             em_ref, msg_ref):
        f32 = jnp.float32
        xe = jnp.concatenate(
            [xe_ref[...], jnp.zeros((blk, 2), f32)], axis=1)
        h = u1_ref[...] + jnp.dot(xe, w1ec_r[...], preferred_element_type=f32)
        h = jnp.maximum(h, 0.0)
        em = jnp.dot(h, w2e_r[...], preferred_element_type=f32) + b2e_r[0:1, :]
        em8 = jnp.concatenate(
            [em, jnp.zeros((blk, 2), f32)], axis=1)
        h2 = (jnp.dot(xd_ref[...], w1na_r[...], preferred_element_type=f32)
              + jnp.dot(em8, w1nb_r[...], preferred_element_type=f32)
              + b1n_r[0:1, :])
        h2 = jnp.maximum(h2, 0.0)
        msg = jnp.dot(h2, w2n_r[...], preferred_element_type=f32) + b2n_r[0:1, :]
        em_ref[...] = em
        msg_ref[...] = msg

    def full(r, c):
        return pl.BlockSpec((r, c), lambda i: (0, 0))

    return pl.pallas_call(
        body,
        grid=grid,
        in_specs=[
            pl.BlockSpec((blk, d), lambda i: (i, 0)),
            pl.BlockSpec((blk, d), lambda i: (i, 0)),
            pl.BlockSpec((blk, de), lambda i: (i, 0)),
            full(8, 32),
            full(32, 6), full(8, 6),
            full(d, 64), full(8, 64), full(8, 64),
            full(64, 32), full(8, 32),
        ],
        out_specs=[
            pl.BlockSpec((blk, 6), lambda i: (i, 0)),
            pl.BlockSpec((blk, 32), lambda i: (i, 0)),
        ],
        out_shape=[
            jax.ShapeDtypeStruct((n_edges, 6), jnp.float32),
            jax.ShapeDtypeStruct((n_edges, 32), jnp.float32),
        ],
        compiler_params=pltpu.CompilerParams(
            dimension_semantics=("arbitrary",),
        ),
    )(u1, xd, x_edge,
      w1ec, w2e, b2e, w1na, w1nb, b1n, w2n, b2n)


def _sc_scatter(msg, dst, n_nodes):
    n_edges, d = msg.shape
    half = n_nodes // 2
    acc_rows = 16 * 3128
    per_tile = n_edges // _NS
    chunk = 80
    n_chunks = per_tile // chunk
    zrows = 512
    mesh = plsc.VectorSubcoreMesh(core_axis_name="c", subcore_axis_name="s")

    @functools.partial(
        pl.kernel,
        out_type=jax.ShapeDtypeStruct((n_nodes, d), jnp.float32),
        mesh=mesh,
        scratch_types=[
            pltpu.VMEM((2, chunk), jnp.int32),
            pltpu.VMEM((2, chunk, d), jnp.float32),
            pltpu.VMEM((zrows, d), jnp.float32),
            pltpu.VMEM_SHARED((acc_rows, d), jnp.float32),
            pltpu.SemaphoreType.DMA,
            pltpu.SemaphoreType.DMA,
        ],
        compiler_params=pltpu.CompilerParams(use_tc_tiling_on_sc=False),
    )
    def scatter_kernel(msg_h, dst_h, out_h, didx, mrows, zbuf, acc, msem, ssem):
        ci = lax.axis_index("c")
        si = lax.axis_index("s")
        node_base = ci * half

        zero16 = jnp.zeros((16,), jnp.float32)

        def zrow(i, carry):
            zbuf[i, pl.ds(0, 16)] = zero16
            zbuf[i, pl.ds(16, 16)] = zero16
            return carry

        lax.fori_loop(0, zrows, zrow, 0)
        tile_base = si * 3128
        for k in range(6):
            pltpu.sync_copy(zbuf.at[pl.ds(0, 512)],
                            acc.at[pl.ds(tile_base + k * 512, 512)])
        pltpu.sync_copy(zbuf.at[pl.ds(0, 56)], acc.at[pl.ds(tile_base + 3072, 56)])
        plsc.subcore_barrier()

        edge_base = si * per_tile

        def fire(j, slot):
            base = edge_base + j * chunk
            pltpu.sync_copy(dst_h.at[pl.ds(base, chunk)], didx.at[slot])
            pltpu.async_copy(msg_h.at[pl.ds(base, chunk)], mrows.at[slot], msem)
            for k in range(chunk // 16):
                v = didx[slot, pl.ds(k * 16, 16)]
                lv = v - node_base
                ok = (lv >= 0) & (lv < half)
                didx[slot, pl.ds(k * 16, 16)] = jnp.where(ok, lv, half)

        fire(0, 0)

        def body(j, carry):
            slot = lax.rem(j, 2)
            nslot = lax.rem(j + 1, 2)

            @pl.when(j >= 1)
            def _():
                pltpu.make_async_copy(
                    mrows.at[nslot], acc.at[didx.at[nslot]], ssem
                ).wait()

            @pl.when(j + 1 < n_chunks)
            def _():
                fire(j + 1, nslot)

            pltpu.make_async_copy(
                msg_h.at[pl.ds(edge_base + j * chunk, chunk)], mrows.at[slot], msem
            ).wait()
            pltpu.async_copy(mrows.at[slot], acc.at[didx.at[slot]], ssem, add=True)
            return carry

        lax.fori_loop(0, n_chunks, body, 0)
        slot_l = (n_chunks - 1) % 2
        pltpu.make_async_copy(
            mrows.at[slot_l], acc.at[didx.at[slot_l]], ssem
        ).wait()
        plsc.subcore_barrier()

        out_base = si * 3128

        def copy_rows(local_r, rows):
            pltpu.sync_copy(acc.at[pl.ds(local_r, rows)], zbuf.at[pl.ds(0, rows)])
            pltpu.sync_copy(zbuf.at[pl.ds(0, rows)],
                            out_h.at[pl.ds(node_base + local_r, rows)])

        for k in range(6):
            copy_rows(out_base + k * 512, 512)

        @pl.when(si < 15)
        def _():
            copy_rows(out_base + 3072, 56)

        @pl.when(si == 15)
        def _():
            copy_rows(out_base + 3072, 8)

    return scatter_kernel(msg, dst)


def kernel(x_node, x_edge, edge_index, b_mask,
           W1e, b1e, W2e, b2e, W1n, b1n, W2n, b2n):
    n_nodes = x_node.shape[0]
    n_edges = x_edge.shape[0]
    src = edge_index[0]
    dst = edge_index[1]

    pad2 = lambda w: jnp.pad(w, ((0, 2), (0, 0)))
    bc8 = lambda b: jnp.broadcast_to(b, (8, b.shape[0]))

    pa, pb = _tc_tables(x_node, W1e[0:32], W1e[32:64], bc8(b1e))

    u1, xd = _sc_gather(x_node, pa, pb, src, dst, b_mask)

    em, msg = _tc_mlp(
        u1, xd, x_edge,
        pad2(W1e[64:70]),
        W2e, bc8(b2e),
        W1n[0:32], pad2(W1n[32:38]), bc8(b1n),
        W2n, bc8(b2n),
    )

    nm = _sc_scatter(msg, dst, n_nodes)
    return (nm, em)


# speedup vs baseline: 4.0453x; 1.6803x over previous
import functools

import jax
import jax.numpy as jnp
from jax import lax
from jax.experimental import pallas as pl
from jax.experimental.pallas import tpu as pltpu
from jax.experimental.pallas import tpu_sc as plsc

_NC = 2
_NS = 16
_NW = _NC * _NS


def _tc_tables(x_node, w1ea, w1eb, b1e):
    n_nodes, d = x_node.shape
    blk = 5000
    grid = (n_nodes // blk,)

    def body(xn_ref, wa_ref, wb_ref, b_ref, pa_ref, pb_ref):
        xn = xn_ref[...]
        f32 = jnp.float32
        pa_ref[...] = (jnp.dot(xn, wa_ref[...], preferred_element_type=f32)
                       + b_ref[0:1, :])
        pb_ref[...] = jnp.dot(xn, wb_ref[...], preferred_element_type=f32)

    return pl.pallas_call(
        body,
        grid=grid,
        in_specs=[
            pl.BlockSpec((blk, d), lambda i: (i, 0)),
            pl.BlockSpec((d, 32), lambda i: (0, 0)),
            pl.BlockSpec((d, 32), lambda i: (0, 0)),
            pl.BlockSpec((8, 32), lambda i: (0, 0)),
        ],
        out_specs=[
            pl.BlockSpec((blk, 32), lambda i: (i, 0)),
            pl.BlockSpec((blk, 32), lambda i: (i, 0)),
        ],
        out_shape=[
            jax.ShapeDtypeStruct((n_nodes, 32), jnp.float32),
            jax.ShapeDtypeStruct((n_nodes, 32), jnp.float32),
        ],
        compiler_params=pltpu.CompilerParams(
            dimension_semantics=("arbitrary",),
        ),
    )(x_node, w1ea, w1eb, b1e)


def _sc_gather(x_node, pa, pb, src, dst, b_mask):
    n_nodes, d = x_node.shape
    n_edges = src.shape[0]
    per_w = n_edges // _NW
    chunk = 80
    n_chunks = per_w // chunk
    ngrp = chunk // 16
    mesh = plsc.VectorSubcoreMesh(core_axis_name="c", subcore_axis_name="s")

    @functools.partial(
        pl.kernel,
        out_type=(
            jax.ShapeDtypeStruct((n_edges, d), jnp.float32),
            jax.ShapeDtypeStruct((n_edges, d), jnp.float32),
        ),
        mesh=mesh,
        scratch_types=[
            pltpu.VMEM((2, chunk), jnp.int32),
            pltpu.VMEM((2, chunk), jnp.int32),
            pltpu.VMEM((2, chunk), jnp.int32),
            pltpu.VMEM((2, chunk, d), jnp.float32),
            pltpu.VMEM((2, chunk, d), jnp.float32),
            pltpu.VMEM((2, chunk, d), jnp.float32),
            pltpu.SemaphoreType.DMA,
            pltpu.SemaphoreType.DMA,
            pltpu.SemaphoreType.DMA,
        ],
        compiler_params=pltpu.CompilerParams(use_tc_tiling_on_sc=False),
    )
    def gather_kernel(xn, pa_h, pb_h, src_h, dst_h, bm_h, u1_out, xd_out,
                      i1b, i2b, mb, ga, gb, gd, gsem, gsem2, osem):
        wid = lax.axis_index("s") * _NC + lax.axis_index("c")
        base0 = wid * per_w

        def fire(j, slot):
            base = base0 + j * chunk
            pltpu.sync_copy(src_h.at[pl.ds(base, chunk)], i1b.at[slot])
            pltpu.sync_copy(dst_h.at[pl.ds(base, chunk)], i2b.at[slot])
            pltpu.sync_copy(bm_h.at[pl.ds(base, chunk)], mb.at[slot])
            for k in range(ngrp):
                sl = pl.ds(k * 16, 16)
                s_v = i1b[slot, sl]
                d_v = i2b[slot, sl]
                fwd = mb[slot, sl] == 0
                i1b[slot, sl] = jnp.where(fwd, s_v, d_v)
                mb[slot, sl] = jnp.where(fwd, d_v, s_v)
            pltpu.async_copy(pa_h.at[i1b.at[slot]], ga.at[slot], gsem)
            pltpu.async_copy(pb_h.at[mb.at[slot]], gb.at[slot], gsem)
            pltpu.async_copy(xn.at[i2b.at[slot]], gd.at[slot], gsem2)

        def drain_gather(slot):
            pltpu.make_async_copy(pa_h.at[i1b.at[slot]], ga.at[slot], gsem).wait()
            pltpu.make_async_copy(pb_h.at[mb.at[slot]], gb.at[slot], gsem).wait()
            pltpu.make_async_copy(xn.at[i2b.at[slot]], gd.at[slot], gsem2).wait()

        fire(0, 0)

        def body(j, carry):
            slot = lax.rem(j, 2)
            nslot = lax.rem(j + 1, 2)

            @pl.when(j >= 1)
            def _():
                base_p = base0 + (j - 1) * chunk
                pltpu.make_async_copy(
                    ga.at[nslot], u1_out.at[pl.ds(base_p, chunk)], osem
                ).wait()
                pltpu.make_async_copy(
                    gd.at[nslot], xd_out.at[pl.ds(base_p, chunk)], osem
                ).wait()

            @pl.when(j + 1 < n_chunks)
            def _():
                fire(j + 1, nslot)

            drain_gather(slot)
            def addrow(r, carry2):
                ga[slot, r, pl.ds(0, 16)] = (
                    ga[slot, r, pl.ds(0, 16)] + gb[slot, r, pl.ds(0, 16)])
                ga[slot, r, pl.ds(16, 16)] = (
                    ga[slot, r, pl.ds(16, 16)] + gb[slot, r, pl.ds(16, 16)])
                return carry2

            lax.fori_loop(0, chunk, addrow, 0)
            base = base0 + j * chunk
            pltpu.async_copy(ga.at[slot], u1_out.at[pl.ds(base, chunk)], osem)
            pltpu.async_copy(gd.at[slot], xd_out.at[pl.ds(base, chunk)], osem)
            return carry

        lax.fori_loop(0, n_chunks, body, 0)
        base_l = base0 + (n_chunks - 1) * chunk
        slot_l = (n_chunks - 1) % 2
        pltpu.make_async_copy(
            ga.at[slot_l], u1_out.at[pl.ds(base_l, chunk)], osem
        ).wait()
        pltpu.make_async_copy(
            gd.at[slot_l], xd_out.at[pl.ds(base_l, chunk)], osem
        ).wait()

    return gather_kernel(x_node, pa, pb, src, dst, b_mask)


def _tc_mlp(u1p, xdp, xep,
            c4, w2e4, b2e4, wna4, wnb4, b1n4, w2n4, b2n4):
    n4 = u1p.shape[0]
    blk4 = 2000
    grid = (n4 // blk4,)

    def body(u1_ref, xd_ref, xe_ref,
             c4_r, w2e4_r, b2e4_r, wna4_r, wnb4_r, b1n4_r, w2n4_r, b2n4_r,
             em_ref, msg_ref):
        f32 = jnp.float32
        h = u1_ref[...] + jnp.dot(xe_ref[...], c4_r[...],
                                  preferred_element_type=f32)
        h = jnp.maximum(h, 0.0)
        em = jnp.dot(h, w2e4_r[...], preferred_element_type=f32) + b2e4_r[0:1, :]
        h2 = (jnp.dot(xd_ref[...], wna4_r[...], preferred_element_type=f32)
              + jnp.dot(em, wnb4_r[...], preferred_element_type=f32)
              + b1n4_r[0:1, :])
        h2 = jnp.maximum(h2, 0.0)
        msg = jnp.dot(h2, w2n4_r[...], preferred_element_type=f32) + b2n4_r[0:1, :]
        em_ref[...] = em
        msg_ref[...] = msg

    def full(r, c):
        return pl.BlockSpec((r, c), lambda i: (0, 0))

    return pl.pallas_call(
        body,
        grid=grid,
        in_specs=[
            pl.BlockSpec((blk4, 128), lambda i: (i, 0)),
            pl.BlockSpec((blk4, 128), lambda i: (i, 0)),
            pl.BlockSpec((blk4, 24), lambda i: (i, 0)),
            full(24, 128),
            full(128, 24), full(8, 24),
            full(128, 256), full(24, 256), full(8, 256),
            full(256, 128), full(8, 128),
        ],
        out_specs=[
            pl.BlockSpec((blk4, 24), lambda i: (i, 0)),
            pl.BlockSpec((blk4, 128), lambda i: (i, 0)),
        ],
        out_shape=[
            jax.ShapeDtypeStruct((n4, 24), jnp.float32),
            jax.ShapeDtypeStruct((n4, 128), jnp.float32),
        ],
        compiler_params=pltpu.CompilerParams(
            dimension_semantics=("arbitrary",),
        ),
    )(u1p, xdp, xep,
      c4, w2e4, b2e4, wna4, wnb4, b1n4, w2n4, b2n4)


def _sc_scatter(msg, dst, n_nodes):
    n_edges, d = msg.shape
    half = n_nodes // 2
    acc_rows = 16 * 3128
    per_tile = n_edges // _NS
    chunk = 80
    n_chunks = per_tile // chunk
    zrows = 512
    mesh = plsc.VectorSubcoreMesh(core_axis_name="c", subcore_axis_name="s")

    @functools.partial(
        pl.kernel,
        out_type=jax.ShapeDtypeStruct((n_nodes, d), jnp.float32),
        mesh=mesh,
        scratch_types=[
            pltpu.VMEM((2, chunk), jnp.int32),
            pltpu.VMEM((2, chunk, d), jnp.float32),
            pltpu.VMEM((zrows, d), jnp.float32),
            pltpu.VMEM_SHARED((acc_rows, d), jnp.float32),
            pltpu.SemaphoreType.DMA,
            pltpu.SemaphoreType.DMA,
        ],
        compiler_params=pltpu.CompilerParams(use_tc_tiling_on_sc=False),
    )
    def scatter_kernel(msg_h, dst_h, out_h, didx, mrows, zbuf, acc, msem, ssem):
        ci = lax.axis_index("c")
        si = lax.axis_index("s")
        node_base = ci * half

        zero16 = jnp.zeros((16,), jnp.float32)

        def zrow(i, carry):
            zbuf[i, pl.ds(0, 16)] = zero16
            zbuf[i, pl.ds(16, 16)] = zero16
            return carry

        lax.fori_loop(0, zrows, zrow, 0)
        tile_base = si * 3128
        for k in range(6):
            pltpu.sync_copy(zbuf.at[pl.ds(0, 512)],
                            acc.at[pl.ds(tile_base + k * 512, 512)])
        pltpu.sync_copy(zbuf.at[pl.ds(0, 56)], acc.at[pl.ds(tile_base + 3072, 56)])
        plsc.subcore_barrier()

        edge_base = si * per_tile

        def fire(j, slot):
            base = edge_base + j * chunk
            pltpu.sync_copy(dst_h.at[pl.ds(base, chunk)], didx.at[slot])
            pltpu.async_copy(msg_h.at[pl.ds(base, chunk)], mrows.at[slot], msem)
            for k in range(chunk // 16):
                v = didx[slot, pl.ds(k * 16, 16)]
                lv = v - node_base
                ok = (lv >= 0) & (lv < half)
                didx[slot, pl.ds(k * 16, 16)] = jnp.where(ok, lv, half)

        fire(0, 0)

        def body(j, carry):
            slot = lax.rem(j, 2)
            nslot = lax.rem(j + 1, 2)

            @pl.when(j >= 1)
            def _():
                pltpu.make_async_copy(
                    mrows.at[nslot], acc.at[didx.at[nslot]], ssem
                ).wait()

            @pl.when(j + 1 < n_chunks)
            def _():
                fire(j + 1, nslot)

            pltpu.make_async_copy(
                msg_h.at[pl.ds(edge_base + j * chunk, chunk)], mrows.at[slot], msem
            ).wait()
            pltpu.async_copy(mrows.at[slot], acc.at[didx.at[slot]], ssem, add=True)
            return carry

        lax.fori_loop(0, n_chunks, body, 0)
        slot_l = (n_chunks - 1) % 2
        pltpu.make_async_copy(
            mrows.at[slot_l], acc.at[didx.at[slot_l]], ssem
        ).wait()
        plsc.subcore_barrier()

        out_base = si * 3128

        def copy_rows(local_r, rows):
            pltpu.sync_copy(acc.at[pl.ds(local_r, rows)], zbuf.at[pl.ds(0, rows)])
            pltpu.sync_copy(zbuf.at[pl.ds(0, rows)],
                            out_h.at[pl.ds(node_base + local_r, rows)])

        for k in range(6):
            copy_rows(out_base + k * 512, 512)

        @pl.when(si < 15)
        def _():
            copy_rows(out_base + 3072, 56)

        @pl.when(si == 15)
        def _():
            copy_rows(out_base + 3072, 8)

    return scatter_kernel(msg, dst)


def kernel(x_node, x_edge, edge_index, b_mask,
           W1e, b1e, W2e, b2e, W1n, b1n, W2n, b2n):
    n_nodes = x_node.shape[0]
    n_edges = x_edge.shape[0]
    src = edge_index[0]
    dst = edge_index[1]

    bc8 = lambda b: jnp.broadcast_to(b, (8, b.shape[0]))
    eye4 = jnp.eye(4, dtype=jnp.float32)
    kron4 = lambda w: jnp.kron(eye4, w)
    tile4 = lambda b: jnp.tile(b, 4)

    pa, pb = _tc_tables(x_node, W1e[0:32], W1e[32:64], bc8(b1e))

    u1, xd = _sc_gather(x_node, pa, pb, src, dst, b_mask)

    em4, msg4 = _tc_mlp(
        u1.reshape(n_edges // 4, 128),
        xd.reshape(n_edges // 4, 128),
        x_edge.reshape(n_edges // 4, 24),
        kron4(W1e[64:70]),
        kron4(W2e), bc8(tile4(b2e)),
        kron4(W1n[0:32]), kron4(W1n[32:38]), bc8(tile4(b1n)),
        kron4(W2n), bc8(tile4(b2n)),
    )

    nm = _sc_scatter(msg4.reshape(n_edges, 32), dst, n_nodes)
    return (nm, em4.reshape(n_edges, 6))


# speedup vs baseline: 5.3815x; 1.3303x over previous
import functools

import jax
import jax.numpy as jnp
from jax import lax
from jax.experimental import pallas as pl
from jax.experimental.pallas import tpu as pltpu
from jax.experimental.pallas import tpu_sc as plsc

_NC = 2
_NS = 16
_NW = _NC * _NS


def _tc_tables(x_node, w1ea, w1eb, b1e):
    n_nodes, d = x_node.shape
    blk = 5000
    grid = (n_nodes // blk,)

    def body(xn_ref, wa_ref, wb_ref, b_ref, pa_ref, pb_ref):
        xn = xn_ref[...]
        f32 = jnp.float32
        pa_ref[...] = (jnp.dot(xn, wa_ref[...], preferred_element_type=f32)
                       + b_ref[0:1, :])
        pb_ref[...] = jnp.dot(xn, wb_ref[...], preferred_element_type=f32)

    return pl.pallas_call(
        body,
        grid=grid,
        in_specs=[
            pl.BlockSpec((blk, d), lambda i: (i, 0)),
            pl.BlockSpec((d, 32), lambda i: (0, 0)),
            pl.BlockSpec((d, 32), lambda i: (0, 0)),
            pl.BlockSpec((8, 32), lambda i: (0, 0)),
        ],
        out_specs=[
            pl.BlockSpec((blk, 32), lambda i: (i, 0)),
            pl.BlockSpec((blk, 32), lambda i: (i, 0)),
        ],
        out_shape=[
            jax.ShapeDtypeStruct((n_nodes, 32), jnp.float32),
            jax.ShapeDtypeStruct((n_nodes, 32), jnp.float32),
        ],
        compiler_params=pltpu.CompilerParams(
            dimension_semantics=("arbitrary",),
        ),
    )(x_node, w1ea, w1eb, b1e)


def _sc_gather(x_node, pa, pb, src, dst, b_mask):
    n_nodes, d = x_node.shape
    n_edges = src.shape[0]
    per_w = n_edges // _NW
    chunk = 80
    n_chunks = per_w // chunk
    ngrp = chunk // 16
    mesh = plsc.VectorSubcoreMesh(core_axis_name="c", subcore_axis_name="s")

    @functools.partial(
        pl.kernel,
        out_type=(
            jax.ShapeDtypeStruct((n_edges, d), jnp.float32),
            jax.ShapeDtypeStruct((n_edges, d), jnp.float32),
        ),
        mesh=mesh,
        scratch_types=[
            pltpu.VMEM((2, chunk), jnp.int32),
            pltpu.VMEM((2, chunk), jnp.int32),
            pltpu.VMEM((2, chunk), jnp.int32),
            pltpu.VMEM((2, chunk, d), jnp.float32),
            pltpu.VMEM((2, chunk, d), jnp.float32),
            pltpu.VMEM((2, chunk, d), jnp.float32),
            pltpu.SemaphoreType.DMA,
            pltpu.SemaphoreType.DMA,
            pltpu.SemaphoreType.DMA,
            pltpu.SemaphoreType.DMA,
        ],
        compiler_params=pltpu.CompilerParams(use_tc_tiling_on_sc=False),
    )
    def gather_kernel(xn, pa_h, pb_h, src_h, dst_h, bm_h, u1_out, xd_out,
                      i1b, i2b, mb, ga, gb, gd, gsem, gsem2, osem, isem):
        wid = lax.axis_index("s") * _NC + lax.axis_index("c")
        base0 = wid * per_w

        def fire_idx(j, slot):
            base = base0 + j * chunk
            pltpu.async_copy(src_h.at[pl.ds(base, chunk)], i1b.at[slot], isem)
            pltpu.async_copy(dst_h.at[pl.ds(base, chunk)], i2b.at[slot], isem)
            pltpu.async_copy(bm_h.at[pl.ds(base, chunk)], mb.at[slot], isem)

        def wait_idx(j, slot):
            base = base0 + j * chunk
            pltpu.make_async_copy(src_h.at[pl.ds(base, chunk)], i1b.at[slot],
                                  isem).wait()
            pltpu.make_async_copy(dst_h.at[pl.ds(base, chunk)], i2b.at[slot],
                                  isem).wait()
            pltpu.make_async_copy(bm_h.at[pl.ds(base, chunk)], mb.at[slot],
                                  isem).wait()

        def fire_gathers(j, slot):
            wait_idx(j, slot)
            for k in range(ngrp):
                sl = pl.ds(k * 16, 16)
                s_v = i1b[slot, sl]
                d_v = i2b[slot, sl]
                fwd = mb[slot, sl] == 0
                i1b[slot, sl] = jnp.where(fwd, s_v, d_v)
                mb[slot, sl] = jnp.where(fwd, d_v, s_v)
            pltpu.async_copy(pa_h.at[i1b.at[slot]], ga.at[slot], gsem)
            pltpu.async_copy(pb_h.at[mb.at[slot]], gb.at[slot], gsem)
            pltpu.async_copy(xn.at[i2b.at[slot]], gd.at[slot], gsem2)

        def drain_gather(slot):
            pltpu.make_async_copy(pa_h.at[i1b.at[slot]], ga.at[slot], gsem).wait()
            pltpu.make_async_copy(pb_h.at[mb.at[slot]], gb.at[slot], gsem).wait()
            pltpu.make_async_copy(xn.at[i2b.at[slot]], gd.at[slot], gsem2).wait()

        fire_idx(0, 0)
        fire_idx(1, 1)
        fire_gathers(0, 0)

        def body(j, carry):
            slot = lax.rem(j, 2)
            nslot = lax.rem(j + 1, 2)

            drain_gather(slot)

            @pl.when(j >= 1)
            def _():
                base_p = base0 + (j - 1) * chunk
                pltpu.make_async_copy(
                    ga.at[nslot], u1_out.at[pl.ds(base_p, chunk)], osem
                ).wait()
                pltpu.make_async_copy(
                    gd.at[nslot], xd_out.at[pl.ds(base_p, chunk)], osem
                ).wait()

            @pl.when(j + 1 < n_chunks)
            def _():
                fire_gathers(j + 1, nslot)

            @pl.when(j + 2 < n_chunks)
            def _():
                fire_idx(j + 2, slot)

            def addrow(g, carry2):
                for k in range(4):
                    r = g * 4 + k
                    ga[slot, r, pl.ds(0, 16)] = (
                        ga[slot, r, pl.ds(0, 16)] + gb[slot, r, pl.ds(0, 16)])
                    ga[slot, r, pl.ds(16, 16)] = (
                        ga[slot, r, pl.ds(16, 16)] + gb[slot, r, pl.ds(16, 16)])
                return carry2

            lax.fori_loop(0, chunk // 4, addrow, 0)
            base = base0 + j * chunk
            pltpu.async_copy(ga.at[slot], u1_out.at[pl.ds(base, chunk)], osem)
            pltpu.async_copy(gd.at[slot], xd_out.at[pl.ds(base, chunk)], osem)
            return carry

        lax.fori_loop(0, n_chunks, body, 0)
        base_l = base0 + (n_chunks - 1) * chunk
        slot_l = (n_chunks - 1) % 2
        pltpu.make_async_copy(
            ga.at[slot_l], u1_out.at[pl.ds(base_l, chunk)], osem
        ).wait()
        pltpu.make_async_copy(
            gd.at[slot_l], xd_out.at[pl.ds(base_l, chunk)], osem
        ).wait()

    return gather_kernel(x_node, pa, pb, src, dst, b_mask)


def _tc_mlp(u1p, xdp, xep,
            c4, w2e4, b2e4, wna4, wnb4, b1n4, w2n4, b2n4):
    n4 = u1p.shape[0]
    blk4 = 2000
    grid = (n4 // blk4,)

    def body(u1_ref, xd_ref, xe_ref,
             c4_r, w2e4_r, b2e4_r, wna4_r, wnb4_r, b1n4_r, w2n4_r, b2n4_r,
             em_ref, msg_ref):
        f32 = jnp.float32
        h = u1_ref[...] + jnp.dot(xe_ref[...], c4_r[...],
                                  preferred_element_type=f32)
        h = jnp.maximum(h, 0.0)
        em = jnp.dot(h, w2e4_r[...], preferred_element_type=f32) + b2e4_r[0:1, :]
        h2 = (jnp.dot(xd_ref[...], wna4_r[...], preferred_element_type=f32)
              + jnp.dot(em, wnb4_r[...], preferred_element_type=f32)
              + b1n4_r[0:1, :])
        h2 = jnp.maximum(h2, 0.0)
        msg = jnp.dot(h2, w2n4_r[...], preferred_element_type=f32) + b2n4_r[0:1, :]
        em_ref[...] = em
        msg_ref[...] = msg

    def full(r, c):
        return pl.BlockSpec((r, c), lambda i: (0, 0))

    return pl.pallas_call(
        body,
        grid=grid,
        in_specs=[
            pl.BlockSpec((blk4, 128), lambda i: (i, 0)),
            pl.BlockSpec((blk4, 128), lambda i: (i, 0)),
            pl.BlockSpec((blk4, 24), lambda i: (i, 0)),
            full(24, 128),
            full(128, 24), full(8, 24),
            full(128, 256), full(24, 256), full(8, 256),
            full(256, 128), full(8, 128),
        ],
        out_specs=[
            pl.BlockSpec((blk4, 24), lambda i: (i, 0)),
            pl.BlockSpec((blk4, 128), lambda i: (i, 0)),
        ],
        out_shape=[
            jax.ShapeDtypeStruct((n4, 24), jnp.float32),
            jax.ShapeDtypeStruct((n4, 128), jnp.float32),
        ],
        compiler_params=pltpu.CompilerParams(
            dimension_semantics=("arbitrary",),
        ),
    )(u1p, xdp, xep,
      c4, w2e4, b2e4, wna4, wnb4, b1n4, w2n4, b2n4)


def _sc_scatter(msg, dst, n_nodes):
    n_edges, d = msg.shape
    half = n_nodes // 2
    acc_rows = 16 * 3128
    per_tile = n_edges // _NS
    chunk = 80
    n_chunks = per_tile // chunk
    zrows = 512
    mesh = plsc.VectorSubcoreMesh(core_axis_name="c", subcore_axis_name="s")

    @functools.partial(
        pl.kernel,
        out_type=jax.ShapeDtypeStruct((n_nodes, d), jnp.float32),
        mesh=mesh,
        scratch_types=[
            pltpu.VMEM((2, chunk), jnp.int32),
            pltpu.VMEM((2, chunk), jnp.int32),
            pltpu.VMEM((2, chunk, d), jnp.float32),
            pltpu.VMEM((zrows, d), jnp.float32),
            pltpu.VMEM_SHARED((acc_rows, d), jnp.float32),
            pltpu.SemaphoreType.DMA,
            pltpu.SemaphoreType.DMA,
            pltpu.SemaphoreType.DMA,
        ],
        compiler_params=pltpu.CompilerParams(use_tc_tiling_on_sc=False),
    )
    def scatter_kernel(msg_h, dst_h, out_h, ridx, didx, mrows, zbuf, acc,
                       msem, ssem, isem):
        ci = lax.axis_index("c")
        si = lax.axis_index("s")
        node_base = ci * half

        zero16 = jnp.zeros((16,), jnp.float32)

        def zrow(i, carry):
            zbuf[i, pl.ds(0, 16)] = zero16
            zbuf[i, pl.ds(16, 16)] = zero16
            return carry

        lax.fori_loop(0, zrows, zrow, 0)
        tile_base = si * 3128
        for k in range(6):
            pltpu.sync_copy(zbuf.at[pl.ds(0, 512)],
                            acc.at[pl.ds(tile_base + k * 512, 512)])
        pltpu.sync_copy(zbuf.at[pl.ds(0, 56)], acc.at[pl.ds(tile_base + 3072, 56)])
        plsc.subcore_barrier()

        edge_base = si * per_tile

        def fire_idx(j, slot):
            base = edge_base + j * chunk
            pltpu.async_copy(dst_h.at[pl.ds(base, chunk)], ridx.at[slot], isem)

        def prep(j, slot):
            base = edge_base + j * chunk
            pltpu.make_async_copy(
                dst_h.at[pl.ds(base, chunk)], ridx.at[slot], isem).wait()
            pltpu.async_copy(msg_h.at[pl.ds(base, chunk)], mrows.at[slot], msem)
            for k in range(chunk // 16):
                v = ridx[slot, pl.ds(k * 16, 16)]
                lv = v - node_base
                ok = (lv >= 0) & (lv < half)
                didx[slot, pl.ds(k * 16, 16)] = jnp.where(ok, lv, half)

        fire_idx(0, 0)
        fire_idx(1, 1)
        prep(0, 0)

        def body(j, carry):
            slot = lax.rem(j, 2)
            nslot = lax.rem(j + 1, 2)

            @pl.when(j >= 1)
            def _():
                pltpu.make_async_copy(
                    mrows.at[nslot], acc.at[didx.at[nslot]], ssem
                ).wait()

            @pl.when(j + 1 < n_chunks)
            def _():
                prep(j + 1, nslot)

            @pl.when(j + 2 < n_chunks)
            def _():
                fire_idx(j + 2, slot)

            pltpu.make_async_copy(
                msg_h.at[pl.ds(edge_base + j * chunk, chunk)], mrows.at[slot], msem
            ).wait()
            pltpu.async_copy(mrows.at[slot], acc.at[didx.at[slot]], ssem, add=True)
            return carry

        lax.fori_loop(0, n_chunks, body, 0)
        slot_l = (n_chunks - 1) % 2
        pltpu.make_async_copy(
            mrows.at[slot_l], acc.at[didx.at[slot_l]], ssem
        ).wait()
        plsc.subcore_barrier()

        out_base = si * 3128

        def copy_rows(local_r, rows):
            pltpu.sync_copy(acc.at[pl.ds(local_r, rows)], zbuf.at[pl.ds(0, rows)])
            pltpu.sync_copy(zbuf.at[pl.ds(0, rows)],
                            out_h.at[pl.ds(node_base + local_r, rows)])

        for k in range(6):
            copy_rows(out_base + k * 512, 512)

        @pl.when(si < 15)
        def _():
            copy_rows(out_base + 3072, 56)

        @pl.when(si == 15)
        def _():
            copy_rows(out_base + 3072, 8)

    return scatter_kernel(msg, dst)


def kernel(x_node, x_edge, edge_index, b_mask,
           W1e, b1e, W2e, b2e, W1n, b1n, W2n, b2n):
    n_nodes = x_node.shape[0]
    n_edges = x_edge.shape[0]
    src = edge_index[0]
    dst = edge_index[1]

    bc8 = lambda b: jnp.broadcast_to(b, (8, b.shape[0]))
    eye4 = jnp.eye(4, dtype=jnp.float32)
    kron4 = lambda w: jnp.kron(eye4, w)
    tile4 = lambda b: jnp.tile(b, 4)

    pa, pb = _tc_tables(x_node, W1e[0:32], W1e[32:64], bc8(b1e))

    u1, xd = _sc_gather(x_node, pa, pb, src, dst, b_mask)

    em4, msg4 = _tc_mlp(
        u1.reshape(n_edges // 4, 128),
        xd.reshape(n_edges // 4, 128),
        x_edge.reshape(n_edges // 4, 24),
        kron4(W1e[64:70]),
        kron4(W2e), bc8(tile4(b2e)),
        kron4(W1n[0:32]), kron4(W1n[32:38]), bc8(tile4(b1n)),
        kron4(W2n), bc8(tile4(b2n)),
    )

    nm = _sc_scatter(msg4.reshape(n_edges, 32), dst, n_nodes)
    return (nm, em4.reshape(n_edges, 6))
